# Initial kernel scaffold; baseline (speedup 1.0000x reference)
import jax
import jax.numpy as jnp
from jax.experimental import pallas as pl


def kernel(x1, edge_index1, batch1, cell, x2, edge_index2, batch2, W_c1, b_c1, W_c2, b_c2, W_c3, b_c3, W_g1, b_g1, W_g2, b_g2, W_r1, b_r1, W_r2, b_r2, W_r3, b_r3, W_f1, b_f1, W_f2, b_f2, W_o, b_o):
    raise NotImplementedError("write your pallas kernel here")


# speedup vs baseline: 1.2732x; 1.2732x over previous
import functools

import jax
import jax.numpy as jnp
from jax.experimental import pallas as pl
from jax.experimental.pallas import tpu as pltpu

N_NODES = 50000
N_EDGES = 800000
N_GRAPHS = 512


def _pad2(a, r, c):
    return jnp.pad(a, ((0, r - a.shape[0]), (0, c - a.shape[1])))


def _pad1(a, n):
    return jnp.pad(a, (0, n - a.shape[0]))


def _mm_body(x_ref, w_ref, b_ref, o_ref, *, relu):
    acc = jnp.dot(x_ref[...], w_ref[...], preferred_element_type=jnp.float32)
    acc = acc + b_ref[...]
    if relu:
        acc = jnp.maximum(acc, 0.0)
    o_ref[...] = acc


def _mm(x, w, b, relu=False, bm=None):
    M, K = x.shape
    _, N = w.shape
    if bm is None:
        bm = M
    grid = (M // bm,)
    return pl.pallas_call(
        functools.partial(_mm_body, relu=relu),
        grid=grid,
        in_specs=[
            pl.BlockSpec((bm, K), lambda i: (i, 0)),
            pl.BlockSpec((K, N), lambda i: (0, 0)),
            pl.BlockSpec((1, N), lambda i: (0, 0)),
        ],
        out_specs=pl.BlockSpec((bm, N), lambda i: (i, 0)),
        out_shape=jax.ShapeDtypeStruct((M, N), jnp.float32),
    )(x, w, b.reshape(1, N))


def _cell_body(x_ref, w_ref, b_ref, o_ref, acc_ref, sq_ref, *, nk):
    k = pl.program_id(0)

    @pl.when(k == 0)
    def _():
        acc_ref[...] = jnp.zeros_like(acc_ref)
        sq_ref[...] = jnp.zeros_like(sq_ref)

    xb = x_ref[...]
    acc_ref[...] += jnp.dot(xb, w_ref[...], preferred_element_type=jnp.float32)
    sq_ref[...] += jnp.sum(xb * xb, axis=1, keepdims=True)

    @pl.when(k == nk - 1)
    def _():
        nrm = jnp.sqrt(sq_ref[...])
        scale = 1.0 / jnp.maximum(nrm, 1e-12)
        o_ref[...] = jnp.maximum(acc_ref[...] * scale + b_ref[...], 0.0)


def _cell_layer1(cell, w, b, bk=2048):
    M, K = cell.shape
    Kp = ((K + bk - 1) // bk) * bk
    cellp = _pad2(cell, M, Kp)
    wp = _pad2(w, Kp, w.shape[1])
    N = w.shape[1]
    nk = Kp // bk
    return pl.pallas_call(
        functools.partial(_cell_body, nk=nk),
        grid=(nk,),
        in_specs=[
            pl.BlockSpec((M, bk), lambda k: (0, k)),
            pl.BlockSpec((bk, N), lambda k: (k, 0)),
            pl.BlockSpec((1, N), lambda k: (0, 0)),
        ],
        out_specs=pl.BlockSpec((M, N), lambda k: (0, 0)),
        out_shape=jax.ShapeDtypeStruct((M, N), jnp.float32),
        scratch_shapes=[
            pltpu.VMEM((M, N), jnp.float32),
            pltpu.VMEM((M, 1), jnp.float32),
        ],
    )(cellp, wp, b.reshape(1, N))


def _head_body(g1_ref, g2_ref, c1_ref, wg1_ref, bg1_ref, wg2_ref, bg2_ref,
               wr2_ref, br2_ref, wr3_ref, br3_ref, wf1_ref, bf1_ref,
               wf2_ref, bf2_ref, wo_ref, bo_ref, o_ref):
    def mm(x, w, b, relu):
        y = jnp.dot(x, w[...], preferred_element_type=jnp.float32) + b[...]
        return jnp.maximum(y, 0.0) if relu else y

    v1 = mm(mm(g1_ref[...], wg1_ref, bg1_ref, True), wg2_ref, bg2_ref, False)
    v2 = mm(mm(g2_ref[...], wg1_ref, bg1_ref, True), wg2_ref, bg2_ref, False)
    c = mm(mm(c1_ref[...], wr2_ref, br2_ref, True), wr3_ref, br3_ref, False)
    xc = jnp.concatenate([v1, v2, c], axis=1)
    h = mm(mm(xc, wf1_ref, bf1_ref, True), wf2_ref, bf2_ref, True)
    o_ref[...] = mm(h, wo_ref, bo_ref, False)


def _head(g1, g2, c1, wg1, bg1, wg2, bg2, wr2, br2, wr3, br3,
          wf1, bf1, wf2, bf2, wo, bo):
    n_out = wo.shape[1]
    args = [g1, g2, c1,
            wg1, bg1.reshape(1, -1), wg2, bg2.reshape(1, -1),
            wr2, br2.reshape(1, -1), wr3, br3.reshape(1, -1),
            wf1, bf1.reshape(1, -1), wf2, bf2.reshape(1, -1),
            wo, bo.reshape(1, -1)]
    return pl.pallas_call(
        _head_body,
        out_shape=jax.ShapeDtypeStruct((N_GRAPHS, n_out), jnp.float32),
    )(*args)


def _gcn_agg(y, row, col, dinv, b, relu=True):
    norm = dinv[row] * dinv[col]
    msg = norm[:, None] * y[row]
    agg = jax.ops.segment_sum(msg, col, num_segments=N_NODES)
    out = agg + (dinv * dinv)[:, None] * y + b
    return jnp.maximum(out, 0.0) if relu else out


def kernel(x1, edge_index1, batch1, cell, x2, edge_index2, batch2,
           W_c1, b_c1, W_c2, b_c2, W_c3, b_c3, W_g1, b_g1, W_g2, b_g2,
           W_r1, b_r1, W_r2, b_r2, W_r3, b_r3, W_f1, b_f1, W_f2, b_f2,
           W_o, b_o):
    wc1 = _pad2(W_c1, 128, 128)
    wc2 = _pad2(W_c2, 128, 256)
    wc3 = _pad2(W_c3, 256, 384)
    bc1 = _pad1(b_c1, 128)
    bc2 = _pad1(b_c2, 256)
    bc3 = _pad1(b_c3, 384)
    wg1 = _pad2(W_g1, 384, 256)
    bg1 = _pad1(b_g1, 256)
    wg2 = _pad2(W_g2, 256, 128)

    def branch(x, ei, batch):
        xp = _pad2(x, N_NODES, 128)
        row, col = ei[0], ei[1]
        deg = 1.0 + jax.ops.segment_sum(
            jnp.ones((N_EDGES,), jnp.float32), col, num_segments=N_NODES)
        dinv = jax.lax.rsqrt(deg)
        h = _mm(xp, wc1, bc1, bm=2000)
        h = _gcn_agg(h, row, col, dinv, bc1)
        h = _mm(h, wc2, bc2, bm=2000)
        h = _gcn_agg(h, row, col, dinv, bc2)
        h = _mm(h, wc3, bc3, bm=2000)
        h = _gcn_agg(h, row, col, dinv, bc3)
        g = jax.ops.segment_max(h, batch, num_segments=N_GRAPHS)
        return g

    g1 = branch(x1, edge_index1, batch1)
    g2 = branch(x2, edge_index2, batch2)

    c1 = _cell_layer1(cell, W_r1, b_r1)

    return _head(g1, g2, c1,
                 wg1, bg1, wg2, b_g2,
                 W_r2, b_r2, W_r3, b_r3,
                 W_f1, b_f1, W_f2, b_f2,
                 W_o, b_o)


# speedup vs baseline: 6.6918x; 5.2558x over previous
import functools

import jax
import jax.numpy as jnp
from jax import lax
from jax.experimental import pallas as pl
from jax.experimental.pallas import tpu as pltpu
from jax.experimental.pallas import tpu_sc as plsc

N_NODES = 50000
N_EDGES = 800000
N_GRAPHS = 512

_MESH = plsc.VectorSubcoreMesh(core_axis_name="c", subcore_axis_name="s")

EPT = N_EDGES // 16
EB = 2000
NB = EPT // EB
J = 25
JW = 80
NP = 50048
RPT = NP // 16
ZR = 136


def _pad2(a, r, c):
    return jnp.pad(a, ((0, r - a.shape[0]), (0, c - a.shape[1])))


def _pad1(a, n):
    return jnp.pad(a, (0, n - a.shape[0]))


DF = 16


def _fill16(ref, val, dtype=jnp.float32):
    v = jnp.full((16,), val, dtype)
    for i in range(ref.shape[0]):
        ref[i, pl.ds(0, 16)] = v


def _sc_agg_call(dchunks):
    @functools.partial(
        pl.kernel,
        out_type=jax.ShapeDtypeStruct((dchunks, NP, DF), jnp.float32),
        mesh=_MESH,
        compiler_params=pltpu.CompilerParams(use_tc_tiling_on_sc=False),
        scratch_types=[
            pltpu.VMEM((EB,), jnp.int32),
            pltpu.VMEM((EB,), jnp.int32),
            pltpu.VMEM((J, JW), jnp.int32),
            pltpu.VMEM((J, JW), jnp.int32),
            pltpu.VMEM((EB, DF), jnp.float32),
            pltpu.VMEM((ZR, DF), jnp.float32),
            pltpu.VMEM_SHARED((NP, DF), jnp.float32),
            pltpu.SemaphoreType.DMA,
            pltpu.SemaphoreType.DMA,
        ],
    )
    def agg(ytab, row, col, out, rowv, colv, gidx, sidx, gbuf, zbuf, acc,
            gsem, ssem):
        c = lax.axis_index("c")
        s = lax.axis_index("s")
        _fill16(zbuf, 0.0)

        def chunk_body(k, _):
            f = c + 2 * k

            def zrow(i, _):
                pltpu.sync_copy(zbuf, acc.at[pl.ds(s * RPT + i * ZR, ZR)])
                return 0

            lax.fori_loop(0, RPT // ZR, zrow, 0)
            plsc.subcore_barrier()

            def batch(bi, _):
                e0 = s * EPT + bi * EB
                pltpu.sync_copy(row.at[pl.ds(e0, EB)], rowv)
                pltpu.sync_copy(col.at[pl.ds(e0, EB)], colv)
                for j in range(J):
                    for q in range(JW // 16):
                        p = j * JW + q * 16
                        r16 = rowv[pl.ds(p, 16)]
                        gidx[j, pl.ds(q * 16, 16)] = r16 * dchunks + f
                        sidx[j, pl.ds(q * 16, 16)] = colv[pl.ds(p, 16)]
                cps = [pltpu.async_copy(ytab.at[gidx.at[j]],
                                        gbuf.at[pl.ds(j * JW, JW)], gsem)
                       for j in range(J)]
                for cp in cps:
                    cp.wait()
                cps = [pltpu.async_copy(gbuf.at[pl.ds(j * JW, JW)],
                                        acc.at[sidx.at[j]], ssem, add=True)
                       for j in range(J)]
                for cp in cps:
                    cp.wait()
                return 0

            lax.fori_loop(0, NB, batch, 0)
            plsc.subcore_barrier()
            pltpu.sync_copy(acc.at[pl.ds(s * RPT, RPT)],
                            out.at[f, pl.ds(s * RPT, RPT)])
            plsc.subcore_barrier()
            return 0

        lax.fori_loop(0, dchunks // 2, chunk_body, 0)

    return agg


_sc_agg = {d: _sc_agg_call(d) for d in (8, 16, 24)}


@functools.partial(
    pl.kernel,
    out_type=jax.ShapeDtypeStruct((2, NP, DF), jnp.float32),
    mesh=_MESH,
    compiler_params=pltpu.CompilerParams(use_tc_tiling_on_sc=False),
    scratch_types=[
        pltpu.VMEM((EB,), jnp.int32),
        pltpu.VMEM((J, JW), jnp.int32),
        pltpu.VMEM((JW, DF), jnp.float32),
        pltpu.VMEM((ZR, DF), jnp.float32),
        pltpu.VMEM_SHARED((NP, DF), jnp.float32),
        pltpu.SemaphoreType.DMA,
    ],
)
def _sc_deg(cols, out, colv, sidx, obuf, zbuf, acc, ssem):
    c = lax.axis_index("c")
    s = lax.axis_index("s")
    _fill16(zbuf, 0.0)
    _fill16(obuf, 1.0)

    def zrow(i, _):
        pltpu.sync_copy(zbuf, acc.at[pl.ds(s * RPT + i * ZR, ZR)])
        return 0

    lax.fori_loop(0, RPT // ZR, zrow, 0)
    plsc.subcore_barrier()

    def batch(bi, _):
        e0 = c * N_EDGES + s * EPT + bi * EB
        pltpu.sync_copy(cols.at[pl.ds(e0, EB)], colv)
        for j in range(J):
            for q in range(JW // 16):
                p = j * JW + q * 16
                sidx[j, pl.ds(q * 16, 16)] = colv[pl.ds(p, 16)]
        cps = [pltpu.async_copy(obuf, acc.at[sidx.at[j]], ssem, add=True)
               for j in range(J)]
        for cp in cps:
            cp.wait()
        return 0

    lax.fori_loop(0, NB, batch, 0)
    plsc.subcore_barrier()
    pltpu.sync_copy(acc.at[pl.ds(s * RPT, RPT)],
                    out.at[c, pl.ds(s * RPT, RPT)])


def _sc_segmax_call(dchunks):
    W = 1568
    SB = 112
    NSB = W // SB

    @functools.partial(
        pl.kernel,
        out_type=jax.ShapeDtypeStruct((dchunks * 32 * N_GRAPHS * DF,), jnp.float32),
        mesh=_MESH,
        compiler_params=pltpu.CompilerParams(use_tc_tiling_on_sc=False),
        scratch_types=[
            pltpu.VMEM((W,), jnp.int32),
            pltpu.VMEM((SB,), jnp.int32),
            pltpu.VMEM((SB, DF), jnp.float32),
            pltpu.VMEM((N_GRAPHS * DF,), jnp.float32),
            pltpu.SemaphoreType.DMA,
        ],
    )
    def segmax(htab, batch, out, batchv, gidx, gbuf, acc, gsem):
        c = lax.axis_index("c")
        s = lax.axis_index("s")
        wid = s * 2 + c
        n0 = jnp.minimum(wid * W, N_NODES - W)
        pltpu.sync_copy(batch.at[pl.ds(n0, W)], batchv)
        iota16 = lax.iota(jnp.int32, 16)

        def chunk_body(f, _):
            def init(i, _):
                acc[pl.ds(i * 16, 16)] = jnp.full((16,), -jnp.inf, jnp.float32)
                return 0

            lax.fori_loop(0, N_GRAPHS * DF // 16, init, 0)

            def subbatch(sb, _):
                r0 = n0 + sb * SB
                for q in range(SB // 16):
                    gidx[pl.ds(q * 16, 16)] = (
                        (r0 + q * 16 + iota16) * dchunks + f)
                pltpu.async_copy(htab.at[gidx], gbuf, gsem).wait()
                for kk in range(0, SB, 16):
                    bv = batchv[pl.ds(sb * SB + kk, 16)]
                    for t in range(16):
                        m = bv[t]
                        a0 = acc[pl.ds(m * DF, 16)]
                        acc[pl.ds(m * DF, 16)] = jnp.maximum(
                            a0, gbuf[kk + t, pl.ds(0, 16)])
                return 0

            lax.fori_loop(0, NSB, subbatch, 0)
            pltpu.sync_copy(acc, out.at[pl.ds((f * 32 + wid) * (N_GRAPHS * DF), N_GRAPHS * DF)])
            return 0

        lax.fori_loop(0, dchunks, chunk_body, 0)

    return segmax


_sc_segmax = {d: _sc_segmax_call(d) for d in (24,)}


def _dinv_body(deg_ref, o_ref):
    o_ref[...] = lax.rsqrt(1.0 + deg_ref[:, :, 0:1])


def _tc_dinv(degacc):
    return pl.pallas_call(
        _dinv_body,
        grid=(2, N_NODES // 2000),
        in_specs=[pl.BlockSpec((1, 2000, DF), lambda b, i: (b, i, 0))],
        out_specs=pl.BlockSpec((1, 2000, 1), lambda b, i: (b, i, 0)),
        out_shape=jax.ShapeDtypeStruct((2, N_NODES, 1), jnp.float32),
    )(degacc)


def _mm_scale_body(x_ref, w_ref, d_ref, o_ref):
    y = jnp.dot(x_ref[...], w_ref[...], preferred_element_type=jnp.float32)
    o_ref[...] = y * d_ref[...]


def _mm_scale(x, w, dinv, bm=2000):
    M, K = x.shape
    _, N = w.shape
    return pl.pallas_call(
        _mm_scale_body,
        grid=(M // bm,),
        in_specs=[
            pl.BlockSpec((bm, K), lambda i: (i, 0)),
            pl.BlockSpec((K, N), lambda i: (0, 0)),
            pl.BlockSpec((bm, 1), lambda i: (i, 0)),
        ],
        out_specs=pl.BlockSpec((bm, N), lambda i: (i, 0)),
        out_shape=jax.ShapeDtypeStruct((M, N), jnp.float32),
    )(x, w, dinv)


def _gcn_x(agg_ref, yt_ref, d_ref, b_ref, dchunks):
    d = d_ref[...]
    parts = []
    for q in range(dchunks):
        z = agg_ref[q] + yt_ref[:, pl.ds(q * DF, DF)]
        parts.append(z)
    z = jnp.concatenate(parts, axis=1)
    return jnp.maximum(z * d + b_ref[...], 0.0)


def _gcn_mm_body(agg_ref, yt_ref, d_ref, b_ref, w_ref, o_ref, *, dchunks):
    h = _gcn_x(agg_ref, yt_ref, d_ref, b_ref, dchunks)
    y = jnp.dot(h, w_ref[...], preferred_element_type=jnp.float32)
    o_ref[...] = y * d_ref[...]


def _gcn_mm(agg3, yt, dinv, b, w, bm=1000):
    M, K = yt.shape
    dchunks = K // DF
    _, N = w.shape
    return pl.pallas_call(
        functools.partial(_gcn_mm_body, dchunks=dchunks),
        grid=(M // bm,),
        in_specs=[
            pl.BlockSpec((dchunks, bm, DF), lambda i: (0, i, 0)),
            pl.BlockSpec((bm, K), lambda i: (i, 0)),
            pl.BlockSpec((bm, 1), lambda i: (i, 0)),
            pl.BlockSpec((1, K), lambda i: (0, 0)),
            pl.BlockSpec((K, N), lambda i: (0, 0)),
        ],
        out_specs=pl.BlockSpec((bm, N), lambda i: (i, 0)),
        out_shape=jax.ShapeDtypeStruct((M, N), jnp.float32),
    )(agg3, yt, dinv, b.reshape(1, K), w)


def _gcn_last_body(agg_ref, yt_ref, d_ref, b_ref, o_ref, *, dchunks):
    o_ref[...] = _gcn_x(agg_ref, yt_ref, d_ref, b_ref, dchunks)


def _gcn_last(agg3, yt, dinv, b, bm=1000):
    M, K = yt.shape
    dchunks = K // DF
    return pl.pallas_call(
        functools.partial(_gcn_last_body, dchunks=dchunks),
        grid=(M // bm,),
        in_specs=[
            pl.BlockSpec((dchunks, bm, DF), lambda i: (0, i, 0)),
            pl.BlockSpec((bm, K), lambda i: (i, 0)),
            pl.BlockSpec((bm, 1), lambda i: (i, 0)),
            pl.BlockSpec((1, K), lambda i: (0, 0)),
        ],
        out_specs=pl.BlockSpec((bm, K), lambda i: (i, 0)),
        out_shape=jax.ShapeDtypeStruct((M, K), jnp.float32),
    )(agg3, yt, dinv, b.reshape(1, K))


def _maxred_body(p_ref, o_ref):
    o_ref[...] = jnp.max(p_ref[...], axis=1).reshape(1, N_GRAPHS, DF)


def _tc_maxreduce(partials, dchunks):
    partials = partials.reshape(dchunks, 32, N_GRAPHS * DF)
    return pl.pallas_call(
        _maxred_body,
        grid=(dchunks,),
        in_specs=[pl.BlockSpec((1, 32, N_GRAPHS * DF), lambda f: (f, 0, 0))],
        out_specs=pl.BlockSpec((1, N_GRAPHS, DF), lambda f: (f, 0, 0)),
        out_shape=jax.ShapeDtypeStruct((dchunks, N_GRAPHS, DF), jnp.float32),
    )(partials)


def _cell_body(x_ref, w_ref, b_ref, o_ref, acc_ref, sq_ref, *, nk):
    k = pl.program_id(0)

    @pl.when(k == 0)
    def _():
        acc_ref[...] = jnp.zeros_like(acc_ref)
        sq_ref[...] = jnp.zeros_like(sq_ref)

    xb = x_ref[...]
    acc_ref[...] += jnp.dot(xb, w_ref[...], preferred_element_type=jnp.float32)
    sq_ref[...] += jnp.sum(xb * xb, axis=1, keepdims=True)

    @pl.when(k == nk - 1)
    def _():
        nrm = jnp.sqrt(sq_ref[...])
        scale = 1.0 / jnp.maximum(nrm, 1e-12)
        o_ref[...] = jnp.maximum(acc_ref[...] * scale + b_ref[...], 0.0)


def _cell_layer1(cell, w, b, bk=2048):
    M, K = cell.shape
    Kp = ((K + bk - 1) // bk) * bk
    cellp = _pad2(cell, M, Kp)
    wp = _pad2(w, Kp, w.shape[1])
    N = w.shape[1]
    nk = Kp // bk
    return pl.pallas_call(
        functools.partial(_cell_body, nk=nk),
        grid=(nk,),
        in_specs=[
            pl.BlockSpec((M, bk), lambda k: (0, k)),
            pl.BlockSpec((bk, N), lambda k: (k, 0)),
            pl.BlockSpec((1, N), lambda k: (0, 0)),
        ],
        out_specs=pl.BlockSpec((M, N), lambda k: (0, 0)),
        out_shape=jax.ShapeDtypeStruct((M, N), jnp.float32),
        scratch_shapes=[
            pltpu.VMEM((M, N), jnp.float32),
            pltpu.VMEM((M, 1), jnp.float32),
        ],
    )(cellp, wp, b.reshape(1, N))


def _head_body(g1_ref, g2_ref, c1_ref, wg1_ref, bg1_ref, wg2_ref, bg2_ref,
               wr2_ref, br2_ref, wr3_ref, br3_ref, wf1_ref, bf1_ref,
               wf2_ref, bf2_ref, wo_ref, bo_ref, o_ref, *, dchunks):
    def mm(x, w, b, relu):
        y = jnp.dot(x, w[...], preferred_element_type=jnp.float32) + b[...]
        return jnp.maximum(y, 0.0) if relu else y

    def flat(g_ref):
        return jnp.concatenate([g_ref[q] for q in range(dchunks)], axis=1)

    v1 = mm(mm(flat(g1_ref), wg1_ref, bg1_ref, True), wg2_ref, bg2_ref, False)
    v2 = mm(mm(flat(g2_ref), wg1_ref, bg1_ref, True), wg2_ref, bg2_ref, False)
    c = mm(mm(c1_ref[...], wr2_ref, br2_ref, True), wr3_ref, br3_ref, False)
    xc = jnp.concatenate([v1, v2, c], axis=1)
    h = mm(mm(xc, wf1_ref, bf1_ref, True), wf2_ref, bf2_ref, True)
    o_ref[...] = mm(h, wo_ref, bo_ref, False)


def _head(g1, g2, c1, wg1, bg1, wg2, bg2, wr2, br2, wr3, br3,
          wf1, bf1, wf2, bf2, wo, bo):
    n_out = wo.shape[1]
    dchunks = g1.shape[0]
    args = [g1, g2, c1,
            wg1, bg1.reshape(1, -1), wg2, bg2.reshape(1, -1),
            wr2, br2.reshape(1, -1), wr3, br3.reshape(1, -1),
            wf1, bf1.reshape(1, -1), wf2, bf2.reshape(1, -1),
            wo, bo.reshape(1, -1)]
    return pl.pallas_call(
        functools.partial(_head_body, dchunks=dchunks),
        out_shape=jax.ShapeDtypeStruct((N_GRAPHS, n_out), jnp.float32),
    )(*args)


def kernel(x1, edge_index1, batch1, cell, x2, edge_index2, batch2,
           W_c1, b_c1, W_c2, b_c2, W_c3, b_c3, W_g1, b_g1, W_g2, b_g2,
           W_r1, b_r1, W_r2, b_r2, W_r3, b_r3, W_f1, b_f1, W_f2, b_f2,
           W_o, b_o):
    wc1 = _pad2(W_c1, 128, 128)
    wc2 = _pad2(W_c2, 128, 256)
    wc3 = _pad2(W_c3, 256, 384)
    bc1 = _pad1(b_c1, 128)
    bc2 = _pad1(b_c2, 256)
    bc3 = _pad1(b_c3, 384)
    wg1 = _pad2(W_g1, 384, 256)
    bg1 = _pad1(b_g1, 256)
    wg2 = _pad2(W_g2, 256, 128)

    cols = jnp.concatenate([edge_index1[1], edge_index2[1]])
    degacc = _sc_deg(cols)
    dinvs = _tc_dinv(degacc)

    def branch(x, ei, batch, dinv):
        xp = _pad2(x, N_NODES, 128)
        row, col = ei[0], ei[1]
        yt1 = _mm_scale(xp, wc1, dinv)
        agg1 = _sc_agg[8](yt1.reshape(N_NODES * 8, DF), row, col)
        yt2 = _gcn_mm(agg1, yt1, dinv, bc1, wc2)
        agg2 = _sc_agg[16](yt2.reshape(N_NODES * 16, DF), row, col)
        yt3 = _gcn_mm(agg2, yt2, dinv, bc2, wc3)
        agg3 = _sc_agg[24](yt3.reshape(N_NODES * 24, DF), row, col)
        h3 = _gcn_last(agg3, yt3, dinv, bc3)
        gp = _sc_segmax[24](h3.reshape(N_NODES * 24, DF), batch)
        return _tc_maxreduce(gp, 24)

    g1 = branch(x1, edge_index1, batch1, dinvs[0])
    g2 = branch(x2, edge_index2, batch2, dinvs[1])

    c1 = _cell_layer1(cell, W_r1, b_r1)

    return _head(g1, g2, c1,
                 wg1, bg1, wg2, b_g2,
                 W_r2, b_r2, W_r3, b_r3,
                 W_f1, b_f1, W_f2, b_f2,
                 W_o, b_o)


# speedup vs baseline: 8.7958x; 1.3144x over previous
import functools

import jax
import jax.numpy as jnp
from jax import lax
from jax.experimental import pallas as pl
from jax.experimental.pallas import tpu as pltpu
from jax.experimental.pallas import tpu_sc as plsc

N_NODES = 50000
N_EDGES = 800000
N_GRAPHS = 512

_MESH = plsc.VectorSubcoreMesh(core_axis_name="c", subcore_axis_name="s")

EPT = N_EDGES // 16
EB = 2000
NB = EPT // EB
J = 25
JW = 80
NP = 50048
RPT = NP // 16
ZR = 136


def _pad2(a, r, c):
    return jnp.pad(a, ((0, r - a.shape[0]), (0, c - a.shape[1])))


def _pad1(a, n):
    return jnp.pad(a, (0, n - a.shape[0]))


DF = 16


def _fill16(ref, val, dtype=jnp.float32):
    v = jnp.full((16,), val, dtype)
    for i in range(ref.shape[0]):
        ref[i, pl.ds(0, 16)] = v


def _sc_agg_call(dchunks):
    @functools.partial(
        pl.kernel,
        out_type=jax.ShapeDtypeStruct((dchunks, NP, DF), jnp.float32),
        mesh=_MESH,
        compiler_params=pltpu.CompilerParams(use_tc_tiling_on_sc=False),
        scratch_types=[
            pltpu.VMEM((EB,), jnp.int32),
            pltpu.VMEM((EB,), jnp.int32),
            pltpu.VMEM((J, JW), jnp.int32),
            pltpu.VMEM((J, JW), jnp.int32),
            pltpu.VMEM((EB, DF), jnp.float32),
            pltpu.VMEM((ZR, DF), jnp.float32),
            pltpu.VMEM_SHARED((NP, DF), jnp.float32),
            pltpu.SemaphoreType.DMA,
            pltpu.SemaphoreType.DMA,
        ],
    )
    def agg(ytab, row, col, out, rowv, colv, gidx, sidx, gbuf, zbuf, acc,
            gsem, ssem):
        c = lax.axis_index("c")
        s = lax.axis_index("s")
        _fill16(zbuf, 0.0)

        def chunk_body(k, _):
            f = c + 2 * k

            def zrow(i, _):
                pltpu.sync_copy(zbuf, acc.at[pl.ds(s * RPT + i * ZR, ZR)])
                return 0

            lax.fori_loop(0, RPT // ZR, zrow, 0)
            plsc.subcore_barrier()

            def batch(bi, _):
                e0 = s * EPT + bi * EB
                pltpu.sync_copy(row.at[pl.ds(e0, EB)], rowv)
                pltpu.sync_copy(col.at[pl.ds(e0, EB)], colv)
                for j in range(J):
                    for q in range(JW // 16):
                        p = j * JW + q * 16
                        r16 = rowv[pl.ds(p, 16)]
                        gidx[j, pl.ds(q * 16, 16)] = r16 * dchunks + f
                        sidx[j, pl.ds(q * 16, 16)] = colv[pl.ds(p, 16)]
                cps = [pltpu.async_copy(ytab.at[gidx.at[j]],
                                        gbuf.at[pl.ds(j * JW, JW)], gsem)
                       for j in range(J)]
                for cp in cps:
                    cp.wait()
                cps = [pltpu.async_copy(gbuf.at[pl.ds(j * JW, JW)],
                                        acc.at[sidx.at[j]], ssem, add=True)
                       for j in range(J)]
                for cp in cps:
                    cp.wait()
                return 0

            lax.fori_loop(0, NB, batch, 0)
            plsc.subcore_barrier()
            pltpu.sync_copy(acc.at[pl.ds(s * RPT, RPT)],
                            out.at[f, pl.ds(s * RPT, RPT)])
            plsc.subcore_barrier()
            return 0

        lax.fori_loop(0, (dchunks + 1 - c) // 2, chunk_body, 0)

    return agg


_sc_agg = {d: _sc_agg_call(d) for d in (5, 10, 20)}


@functools.partial(
    pl.kernel,
    out_type=jax.ShapeDtypeStruct((2, NP, DF), jnp.float32),
    mesh=_MESH,
    compiler_params=pltpu.CompilerParams(use_tc_tiling_on_sc=False),
    scratch_types=[
        pltpu.VMEM((EB,), jnp.int32),
        pltpu.VMEM((J, JW), jnp.int32),
        pltpu.VMEM((JW, DF), jnp.float32),
        pltpu.VMEM((ZR, DF), jnp.float32),
        pltpu.VMEM_SHARED((NP, DF), jnp.float32),
        pltpu.SemaphoreType.DMA,
    ],
)
def _sc_deg(cols, out, colv, sidx, obuf, zbuf, acc, ssem):
    c = lax.axis_index("c")
    s = lax.axis_index("s")
    _fill16(zbuf, 0.0)
    _fill16(obuf, 1.0)

    def zrow(i, _):
        pltpu.sync_copy(zbuf, acc.at[pl.ds(s * RPT + i * ZR, ZR)])
        return 0

    lax.fori_loop(0, RPT // ZR, zrow, 0)
    plsc.subcore_barrier()

    def batch(bi, _):
        e0 = c * N_EDGES + s * EPT + bi * EB
        pltpu.sync_copy(cols.at[pl.ds(e0, EB)], colv)
        for j in range(J):
            for q in range(JW // 16):
                p = j * JW + q * 16
                sidx[j, pl.ds(q * 16, 16)] = colv[pl.ds(p, 16)]
        cps = [pltpu.async_copy(obuf, acc.at[sidx.at[j]], ssem, add=True)
               for j in range(J)]
        for cp in cps:
            cp.wait()
        return 0

    lax.fori_loop(0, NB, batch, 0)
    plsc.subcore_barrier()
    pltpu.sync_copy(acc.at[pl.ds(s * RPT, RPT)],
                    out.at[c, pl.ds(s * RPT, RPT)])


def _sc_segmax_call(dchunks):
    W = 1568
    SB = 112
    NSB = W // SB

    @functools.partial(
        pl.kernel,
        out_type=jax.ShapeDtypeStruct((dchunks * 32 * N_GRAPHS * DF,), jnp.float32),
        mesh=_MESH,
        compiler_params=pltpu.CompilerParams(use_tc_tiling_on_sc=False),
        scratch_types=[
            pltpu.VMEM((W,), jnp.int32),
            pltpu.VMEM((SB,), jnp.int32),
            pltpu.VMEM((SB, DF), jnp.float32),
            pltpu.VMEM((N_GRAPHS * DF,), jnp.float32),
            pltpu.SemaphoreType.DMA,
        ],
    )
    def segmax(htab, batch, out, batchv, gidx, gbuf, acc, gsem):
        c = lax.axis_index("c")
        s = lax.axis_index("s")
        wid = s * 2 + c
        n0 = jnp.minimum(wid * W, N_NODES - W)
        pltpu.sync_copy(batch.at[pl.ds(n0, W)], batchv)
        iota16 = lax.iota(jnp.int32, 16)

        def chunk_body(f, _):
            def init(i, _):
                acc[pl.ds(i * 16, 16)] = jnp.full((16,), -jnp.inf, jnp.float32)
                return 0

            lax.fori_loop(0, N_GRAPHS * DF // 16, init, 0)

            def subbatch(sb, _):
                r0 = n0 + sb * SB
                for q in range(SB // 16):
                    gidx[pl.ds(q * 16, 16)] = (
                        (r0 + q * 16 + iota16) * dchunks + f)
                pltpu.async_copy(htab.at[gidx], gbuf, gsem).wait()
                for kk in range(0, SB, 16):
                    bv = batchv[pl.ds(sb * SB + kk, 16)]
                    for t in range(16):
                        m = bv[t]
                        a0 = acc[pl.ds(m * DF, 16)]
                        acc[pl.ds(m * DF, 16)] = jnp.maximum(
                            a0, gbuf[kk + t, pl.ds(0, 16)])
                return 0

            lax.fori_loop(0, NSB, subbatch, 0)
            pltpu.sync_copy(acc, out.at[pl.ds((f * 32 + wid) * (N_GRAPHS * DF), N_GRAPHS * DF)])
            return 0

        lax.fori_loop(0, dchunks, chunk_body, 0)

    return segmax


_sc_segmax = {d: _sc_segmax_call(d) for d in (20,)}


def _dinv_body(deg_ref, o_ref):
    o_ref[...] = lax.rsqrt(1.0 + deg_ref[:, :, 0:1])


def _tc_dinv(degacc):
    return pl.pallas_call(
        _dinv_body,
        grid=(2, N_NODES // 2000),
        in_specs=[pl.BlockSpec((1, 2000, DF), lambda b, i: (b, i, 0))],
        out_specs=pl.BlockSpec((1, 2000, 1), lambda b, i: (b, i, 0)),
        out_shape=jax.ShapeDtypeStruct((2, N_NODES, 1), jnp.float32),
    )(degacc)


def _mm_scale_body(x_ref, w_ref, d_ref, o_ref):
    y = jnp.dot(x_ref[...], w_ref[...], preferred_element_type=jnp.float32)
    o_ref[...] = y * d_ref[...]


def _mm_scale(x, w, dinv, bm=2000):
    M, K = x.shape
    _, N = w.shape
    return pl.pallas_call(
        _mm_scale_body,
        grid=(M // bm,),
        in_specs=[
            pl.BlockSpec((bm, K), lambda i: (i, 0)),
            pl.BlockSpec((K, N), lambda i: (0, 0)),
            pl.BlockSpec((bm, 1), lambda i: (i, 0)),
        ],
        out_specs=pl.BlockSpec((bm, N), lambda i: (i, 0)),
        out_shape=jax.ShapeDtypeStruct((M, N), jnp.float32),
    )(x, w, dinv)


def _gcn_x(agg_ref, yt_ref, d_ref, b_ref, dchunks):
    d = d_ref[...]
    parts = []
    for q in range(dchunks):
        z = agg_ref[q] + yt_ref[:, pl.ds(q * DF, DF)]
        parts.append(z)
    z = jnp.concatenate(parts, axis=1)
    return jnp.maximum(z * d + b_ref[...], 0.0)


def _gcn_mm_body(agg_ref, yt_ref, d_ref, b_ref, w_ref, o_ref, *, dchunks):
    h = _gcn_x(agg_ref, yt_ref, d_ref, b_ref, dchunks)
    y = jnp.dot(h, w_ref[...], preferred_element_type=jnp.float32)
    o_ref[...] = y * d_ref[...]


def _gcn_mm(agg3, yt, dinv, b, w, bm=1000):
    M, K = yt.shape
    dchunks = K // DF
    _, N = w.shape
    return pl.pallas_call(
        functools.partial(_gcn_mm_body, dchunks=dchunks),
        grid=(M // bm,),
        in_specs=[
            pl.BlockSpec((dchunks, bm, DF), lambda i: (0, i, 0)),
            pl.BlockSpec((bm, K), lambda i: (i, 0)),
            pl.BlockSpec((bm, 1), lambda i: (i, 0)),
            pl.BlockSpec((1, K), lambda i: (0, 0)),
            pl.BlockSpec((K, N), lambda i: (0, 0)),
        ],
        out_specs=pl.BlockSpec((bm, N), lambda i: (i, 0)),
        out_shape=jax.ShapeDtypeStruct((M, N), jnp.float32),
    )(agg3, yt, dinv, b.reshape(1, K), w)


def _gcn_last_body(agg_ref, yt_ref, d_ref, b_ref, o_ref, *, dchunks):
    o_ref[...] = _gcn_x(agg_ref, yt_ref, d_ref, b_ref, dchunks)


def _gcn_last(agg3, yt, dinv, b, bm=1000):
    M, K = yt.shape
    dchunks = K // DF
    return pl.pallas_call(
        functools.partial(_gcn_last_body, dchunks=dchunks),
        grid=(M // bm,),
        in_specs=[
            pl.BlockSpec((dchunks, bm, DF), lambda i: (0, i, 0)),
            pl.BlockSpec((bm, K), lambda i: (i, 0)),
            pl.BlockSpec((bm, 1), lambda i: (i, 0)),
            pl.BlockSpec((1, K), lambda i: (0, 0)),
        ],
        out_specs=pl.BlockSpec((bm, K), lambda i: (i, 0)),
        out_shape=jax.ShapeDtypeStruct((M, K), jnp.float32),
    )(agg3, yt, dinv, b.reshape(1, K))


def _maxred_body(p_ref, o_ref):
    o_ref[...] = jnp.max(p_ref[...], axis=1).reshape(1, N_GRAPHS, DF)


def _tc_maxreduce(partials, dchunks):
    partials = partials.reshape(dchunks, 32, N_GRAPHS * DF)
    return pl.pallas_call(
        _maxred_body,
        grid=(dchunks,),
        in_specs=[pl.BlockSpec((1, 32, N_GRAPHS * DF), lambda f: (f, 0, 0))],
        out_specs=pl.BlockSpec((1, N_GRAPHS, DF), lambda f: (f, 0, 0)),
        out_shape=jax.ShapeDtypeStruct((dchunks, N_GRAPHS, DF), jnp.float32),
    )(partials)


def _cell_body(x_ref, w_ref, b_ref, o_ref, acc_ref, sq_ref, *, nk):
    k = pl.program_id(0)

    @pl.when(k == 0)
    def _():
        acc_ref[...] = jnp.zeros_like(acc_ref)
        sq_ref[...] = jnp.zeros_like(sq_ref)

    xb = x_ref[...]
    acc_ref[...] += jnp.dot(xb, w_ref[...], preferred_element_type=jnp.float32)
    sq_ref[...] += jnp.sum(xb * xb, axis=1, keepdims=True)

    @pl.when(k == nk - 1)
    def _():
        nrm = jnp.sqrt(sq_ref[...])
        scale = 1.0 / jnp.maximum(nrm, 1e-12)
        o_ref[...] = jnp.maximum(acc_ref[...] * scale + b_ref[...], 0.0)


def _cell_layer1(cell, w, b, bk=2048):
    M, K = cell.shape
    Kp = ((K + bk - 1) // bk) * bk
    cellp = _pad2(cell, M, Kp)
    wp = _pad2(w, Kp, w.shape[1])
    N = w.shape[1]
    nk = Kp // bk
    return pl.pallas_call(
        functools.partial(_cell_body, nk=nk),
        grid=(nk,),
        in_specs=[
            pl.BlockSpec((M, bk), lambda k: (0, k)),
            pl.BlockSpec((bk, N), lambda k: (k, 0)),
            pl.BlockSpec((1, N), lambda k: (0, 0)),
        ],
        out_specs=pl.BlockSpec((M, N), lambda k: (0, 0)),
        out_shape=jax.ShapeDtypeStruct((M, N), jnp.float32),
        scratch_shapes=[
            pltpu.VMEM((M, N), jnp.float32),
            pltpu.VMEM((M, 1), jnp.float32),
        ],
    )(cellp, wp, b.reshape(1, N))


def _head_body(g1_ref, g2_ref, c1_ref, wg1_ref, bg1_ref, wg2_ref, bg2_ref,
               wr2_ref, br2_ref, wr3_ref, br3_ref, wf1_ref, bf1_ref,
               wf2_ref, bf2_ref, wo_ref, bo_ref, o_ref, *, dchunks):
    def mm(x, w, b, relu):
        y = jnp.dot(x, w[...], preferred_element_type=jnp.float32) + b[...]
        return jnp.maximum(y, 0.0) if relu else y

    def flat(g_ref):
        return jnp.concatenate([g_ref[q] for q in range(dchunks)], axis=1)

    v1 = mm(mm(flat(g1_ref), wg1_ref, bg1_ref, True), wg2_ref, bg2_ref, False)
    v2 = mm(mm(flat(g2_ref), wg1_ref, bg1_ref, True), wg2_ref, bg2_ref, False)
    c = mm(mm(c1_ref[...], wr2_ref, br2_ref, True), wr3_ref, br3_ref, False)
    xc = jnp.concatenate([v1, v2, c], axis=1)
    h = mm(mm(xc, wf1_ref, bf1_ref, True), wf2_ref, bf2_ref, True)
    o_ref[...] = mm(h, wo_ref, bo_ref, False)


def _head(g1, g2, c1, wg1, bg1, wg2, bg2, wr2, br2, wr3, br3,
          wf1, bf1, wf2, bf2, wo, bo):
    n_out = wo.shape[1]
    dchunks = g1.shape[0]
    args = [g1, g2, c1,
            wg1, bg1.reshape(1, -1), wg2, bg2.reshape(1, -1),
            wr2, br2.reshape(1, -1), wr3, br3.reshape(1, -1),
            wf1, bf1.reshape(1, -1), wf2, bf2.reshape(1, -1),
            wo, bo.reshape(1, -1)]
    return pl.pallas_call(
        functools.partial(_head_body, dchunks=dchunks),
        out_shape=jax.ShapeDtypeStruct((N_GRAPHS, n_out), jnp.float32),
    )(*args)


def kernel(x1, edge_index1, batch1, cell, x2, edge_index2, batch2,
           W_c1, b_c1, W_c2, b_c2, W_c3, b_c3, W_g1, b_g1, W_g2, b_g2,
           W_r1, b_r1, W_r2, b_r2, W_r3, b_r3, W_f1, b_f1, W_f2, b_f2,
           W_o, b_o):
    wc1 = _pad2(W_c1, 128, 80)
    wc2 = _pad2(W_c2, 80, 160)
    wc3 = _pad2(W_c3, 160, 320)
    bc1 = _pad1(b_c1, 80)
    bc2 = _pad1(b_c2, 160)
    bc3 = _pad1(b_c3, 320)
    wg1 = _pad2(W_g1, 320, 160)
    bg1 = _pad1(b_g1, 160)
    wg2 = _pad2(W_g2, 160, 128)

    cols = jnp.concatenate([edge_index1[1], edge_index2[1]])
    degacc = _sc_deg(cols)
    dinvs = _tc_dinv(degacc)

    def branch(x, ei, batch, dinv):
        xp = _pad2(x, N_NODES, 128)
        row, col = ei[0], ei[1]
        yt1 = _mm_scale(xp, wc1, dinv)
        agg1 = _sc_agg[5](yt1.reshape(N_NODES * 5, DF), row, col)
        yt2 = _gcn_mm(agg1, yt1, dinv, bc1, wc2)
        agg2 = _sc_agg[10](yt2.reshape(N_NODES * 10, DF), row, col)
        yt3 = _gcn_mm(agg2, yt2, dinv, bc2, wc3)
        agg3 = _sc_agg[20](yt3.reshape(N_NODES * 20, DF), row, col)
        h3 = _gcn_last(agg3, yt3, dinv, bc3)
        gp = _sc_segmax[20](h3.reshape(N_NODES * 20, DF), batch)
        return _tc_maxreduce(gp, 20)

    g1 = branch(x1, edge_index1, batch1, dinvs[0])
    g2 = branch(x2, edge_index2, batch2, dinvs[1])

    c1 = _cell_layer1(cell, W_r1, b_r1)

    return _head(g1, g2, c1,
                 wg1, bg1, wg2, b_g2,
                 W_r2, b_r2, W_r3, b_r3,
                 W_f1, b_f1, W_f2, b_f2,
                 W_o, b_o)


# speedup vs baseline: 11.7110x; 1.3314x over previous
import functools

import jax
import jax.numpy as jnp
from jax import lax
from jax.experimental import pallas as pl
from jax.experimental.pallas import tpu as pltpu
from jax.experimental.pallas import tpu_sc as plsc

N_NODES = 50000
N_EDGES = 800000
N_GRAPHS = 512

_MESH = plsc.VectorSubcoreMesh(core_axis_name="c", subcore_axis_name="s")

EPT = N_EDGES // 16
EB = 2000
NB = EPT // EB
J = 25
JW = 80
NP = 50048
RPT = NP // 16
ZR = 136


def _pad2(a, r, c):
    return jnp.pad(a, ((0, r - a.shape[0]), (0, c - a.shape[1])))


def _pad1(a, n):
    return jnp.pad(a, (0, n - a.shape[0]))


DF = 16


def _fill16(ref, val, dtype=jnp.float32):
    v = jnp.full((16,), val, dtype)
    for i in range(ref.shape[0]):
        ref[i, pl.ds(0, 16)] = v


def _sc_agg_call(dchunks):
    @functools.partial(
        pl.kernel,
        out_type=jax.ShapeDtypeStruct((dchunks, NP, DF), jnp.float32),
        mesh=_MESH,
        compiler_params=pltpu.CompilerParams(use_tc_tiling_on_sc=False),
        scratch_types=[
            pltpu.VMEM((EB,), jnp.int32),
            pltpu.VMEM((EB,), jnp.int32),
            pltpu.VMEM((J, JW), jnp.int32),
            pltpu.VMEM((J, JW), jnp.int32),
            pltpu.VMEM((EB, DF), jnp.float32),
            pltpu.VMEM((J, JW), jnp.int32),
            pltpu.VMEM((J, JW), jnp.int32),
            pltpu.VMEM((EB, DF), jnp.float32),
            pltpu.VMEM((ZR, DF), jnp.float32),
            pltpu.VMEM_SHARED((NP, DF), jnp.float32),
            pltpu.SemaphoreType.DMA,
            pltpu.SemaphoreType.DMA,
            pltpu.SemaphoreType.DMA,
            pltpu.SemaphoreType.DMA,
        ],
    )
    def agg(ytab, row, col, out,
            rowvA, colvA, gidxA, sidxA, gbufA,
            gidxB, sidxB, gbufB,
            zbuf, acc, gsemA, ssemA, gsemB, ssemB):
        c = lax.axis_index("c")
        s = lax.axis_index("s")
        _fill16(zbuf, 0.0)

        A = (rowvA, colvA, gidxA, sidxA, gbufA, gsemA, ssemA)
        B = (rowvA, colvA, gidxB, sidxB, gbufB, gsemB, ssemB)

        def chunk_body(k, _):
            f = c + 2 * k

            def zrow(i, _):
                pltpu.sync_copy(zbuf, acc.at[pl.ds(s * RPT + i * ZR, ZR)])
                return 0

            lax.fori_loop(0, RPT // ZR, zrow, 0)
            plsc.subcore_barrier()

            def build(st, bi):
                rowv, colv, gidx, sidx = st[0], st[1], st[2], st[3]
                e0 = s * EPT + bi * EB
                pltpu.sync_copy(row.at[pl.ds(e0, EB)], rowv)
                pltpu.sync_copy(col.at[pl.ds(e0, EB)], colv)
                for j in range(J):
                    for q in range(JW // 16):
                        pp = j * JW + q * 16
                        r16 = rowv[pl.ds(pp, 16)]
                        gidx[j, pl.ds(q * 16, 16)] = r16 * dchunks + f
                        sidx[j, pl.ds(q * 16, 16)] = colv[pl.ds(pp, 16)]

            def fire_g(st):
                for j in range(J):
                    pltpu.async_copy(ytab.at[st[2].at[j]],
                                     st[4].at[pl.ds(j * JW, JW)], st[5])

            def drain_g(st):
                for j in range(J):
                    pltpu.make_async_copy(
                        ytab.at[st[2].at[j]],
                        st[4].at[pl.ds(j * JW, JW)], st[5]).wait()

            def fire_s(st):
                for j in range(J):
                    pltpu.async_copy(st[4].at[pl.ds(j * JW, JW)],
                                     acc.at[st[3].at[j]], st[6], add=True)

            def drain_s(st):
                for j in range(J):
                    pltpu.make_async_copy(
                        st[4].at[pl.ds(j * JW, JW)],
                        acc.at[st[3].at[j]], st[6]).wait()

            build(A, 0)
            fire_g(A)

            def pair(m, _):
                build(B, 2 * m + 1)
                drain_g(A)
                fire_s(A)
                fire_g(B)
                drain_s(A)
                build(A, 2 * m + 2)
                drain_g(B)
                fire_s(B)
                fire_g(A)
                drain_s(B)
                return 0

            lax.fori_loop(0, (NB - 1) // 2, pair, 0)
            drain_g(A)
            fire_s(A)
            drain_s(A)

            plsc.subcore_barrier()
            pltpu.sync_copy(acc.at[pl.ds(s * RPT, RPT)],
                            out.at[f, pl.ds(s * RPT, RPT)])
            plsc.subcore_barrier()
            return 0

        lax.fori_loop(0, (dchunks + 1 - c) // 2, chunk_body, 0)

    return agg


_sc_agg = {d: _sc_agg_call(d) for d in (5, 10, 20)}


@functools.partial(
    pl.kernel,
    out_type=jax.ShapeDtypeStruct((2, NP, DF), jnp.float32),
    mesh=_MESH,
    compiler_params=pltpu.CompilerParams(use_tc_tiling_on_sc=False),
    scratch_types=[
        pltpu.VMEM((EB,), jnp.int32),
        pltpu.VMEM((J, JW), jnp.int32),
        pltpu.VMEM((JW, DF), jnp.float32),
        pltpu.VMEM((ZR, DF), jnp.float32),
        pltpu.VMEM_SHARED((NP, DF), jnp.float32),
        pltpu.SemaphoreType.DMA,
    ],
)
def _sc_deg(cols, out, colv, sidx, obuf, zbuf, acc, ssem):
    c = lax.axis_index("c")
    s = lax.axis_index("s")
    _fill16(zbuf, 0.0)
    _fill16(obuf, 1.0)

    def zrow(i, _):
        pltpu.sync_copy(zbuf, acc.at[pl.ds(s * RPT + i * ZR, ZR)])
        return 0

    lax.fori_loop(0, RPT // ZR, zrow, 0)
    plsc.subcore_barrier()

    def batch(bi, _):
        e0 = c * N_EDGES + s * EPT + bi * EB
        pltpu.sync_copy(cols.at[pl.ds(e0, EB)], colv)
        for j in range(J):
            for q in range(JW // 16):
                p = j * JW + q * 16
                sidx[j, pl.ds(q * 16, 16)] = colv[pl.ds(p, 16)]
        cps = [pltpu.async_copy(obuf, acc.at[sidx.at[j]], ssem, add=True)
               for j in range(J)]
        for cp in cps:
            cp.wait()
        return 0

    lax.fori_loop(0, NB, batch, 0)
    plsc.subcore_barrier()
    pltpu.sync_copy(acc.at[pl.ds(s * RPT, RPT)],
                    out.at[c, pl.ds(s * RPT, RPT)])


def _sc_segmax_call(dchunks):
    W = 1568
    SB = 112
    NSB = W // SB

    @functools.partial(
        pl.kernel,
        out_type=jax.ShapeDtypeStruct((dchunks * 32 * N_GRAPHS * DF,), jnp.float32),
        mesh=_MESH,
        compiler_params=pltpu.CompilerParams(use_tc_tiling_on_sc=False),
        scratch_types=[
            pltpu.VMEM((W,), jnp.int32),
            pltpu.VMEM((SB,), jnp.int32),
            pltpu.VMEM((SB, DF), jnp.float32),
            pltpu.VMEM((N_GRAPHS * DF,), jnp.float32),
            pltpu.SemaphoreType.DMA,
        ],
    )
    def segmax(htab, batch, out, batchv, gidx, gbuf, acc, gsem):
        c = lax.axis_index("c")
        s = lax.axis_index("s")
        wid = s * 2 + c
        n0 = jnp.minimum(wid * W, N_NODES - W)
        pltpu.sync_copy(batch.at[pl.ds(n0, W)], batchv)
        iota16 = lax.iota(jnp.int32, 16)

        def chunk_body(f, _):
            def init(i, _):
                acc[pl.ds(i * 16, 16)] = jnp.full((16,), -jnp.inf, jnp.float32)
                return 0

            lax.fori_loop(0, N_GRAPHS * DF // 16, init, 0)

            def subbatch(sb, _):
                r0 = n0 + sb * SB
                for q in range(SB // 16):
                    gidx[pl.ds(q * 16, 16)] = (
                        (r0 + q * 16 + iota16) * dchunks + f)
                pltpu.async_copy(htab.at[gidx], gbuf, gsem).wait()
                for kk in range(0, SB, 16):
                    bv = batchv[pl.ds(sb * SB + kk, 16)]
                    for t in range(16):
                        m = bv[t]
                        a0 = acc[pl.ds(m * DF, 16)]
                        acc[pl.ds(m * DF, 16)] = jnp.maximum(
                            a0, gbuf[kk + t, pl.ds(0, 16)])
                return 0

            lax.fori_loop(0, NSB, subbatch, 0)
            pltpu.sync_copy(acc, out.at[pl.ds((f * 32 + wid) * (N_GRAPHS * DF), N_GRAPHS * DF)])
            return 0

        lax.fori_loop(0, dchunks, chunk_body, 0)

    return segmax


_sc_segmax = {d: _sc_segmax_call(d) for d in (20,)}


def _dinv_body(deg_ref, o_ref):
    o_ref[...] = lax.rsqrt(1.0 + deg_ref[:, :, 0:1])


def _tc_dinv(degacc):
    return pl.pallas_call(
        _dinv_body,
        grid=(2, N_NODES // 2000),
        in_specs=[pl.BlockSpec((1, 2000, DF), lambda b, i: (b, i, 0))],
        out_specs=pl.BlockSpec((1, 2000, 1), lambda b, i: (b, i, 0)),
        out_shape=jax.ShapeDtypeStruct((2, N_NODES, 1), jnp.float32),
    )(degacc)


def _mm_scale_body(x_ref, w_ref, d_ref, o_ref):
    y = jnp.dot(x_ref[...], w_ref[...], preferred_element_type=jnp.float32)
    o_ref[...] = y * d_ref[...]


def _mm_scale(x, w, dinv, bm=2000):
    M, K = x.shape
    _, N = w.shape
    return pl.pallas_call(
        _mm_scale_body,
        grid=(M // bm,),
        in_specs=[
            pl.BlockSpec((bm, K), lambda i: (i, 0)),
            pl.BlockSpec((K, N), lambda i: (0, 0)),
            pl.BlockSpec((bm, 1), lambda i: (i, 0)),
        ],
        out_specs=pl.BlockSpec((bm, N), lambda i: (i, 0)),
        out_shape=jax.ShapeDtypeStruct((M, N), jnp.float32),
    )(x, w, dinv)


def _gcn_x(agg_ref, yt_ref, d_ref, b_ref, dchunks):
    d = d_ref[...]
    parts = []
    for q in range(dchunks):
        z = agg_ref[q] + yt_ref[:, pl.ds(q * DF, DF)]
        parts.append(z)
    z = jnp.concatenate(parts, axis=1)
    return jnp.maximum(z * d + b_ref[...], 0.0)


def _gcn_mm_body(agg_ref, yt_ref, d_ref, b_ref, w_ref, o_ref, *, dchunks):
    h = _gcn_x(agg_ref, yt_ref, d_ref, b_ref, dchunks)
    y = jnp.dot(h, w_ref[...], preferred_element_type=jnp.float32)
    o_ref[...] = y * d_ref[...]


def _gcn_mm(agg3, yt, dinv, b, w, bm=1000):
    M, K = yt.shape
    dchunks = K // DF
    _, N = w.shape
    return pl.pallas_call(
        functools.partial(_gcn_mm_body, dchunks=dchunks),
        grid=(M // bm,),
        in_specs=[
            pl.BlockSpec((dchunks, bm, DF), lambda i: (0, i, 0)),
            pl.BlockSpec((bm, K), lambda i: (i, 0)),
            pl.BlockSpec((bm, 1), lambda i: (i, 0)),
            pl.BlockSpec((1, K), lambda i: (0, 0)),
            pl.BlockSpec((K, N), lambda i: (0, 0)),
        ],
        out_specs=pl.BlockSpec((bm, N), lambda i: (i, 0)),
        out_shape=jax.ShapeDtypeStruct((M, N), jnp.float32),
    )(agg3, yt, dinv, b.reshape(1, K), w)


def _gcn_last_body(agg_ref, yt_ref, d_ref, b_ref, o_ref, *, dchunks):
    o_ref[...] = _gcn_x(agg_ref, yt_ref, d_ref, b_ref, dchunks)


def _gcn_last(agg3, yt, dinv, b, bm=1000):
    M, K = yt.shape
    dchunks = K // DF
    return pl.pallas_call(
        functools.partial(_gcn_last_body, dchunks=dchunks),
        grid=(M // bm,),
        in_specs=[
            pl.BlockSpec((dchunks, bm, DF), lambda i: (0, i, 0)),
            pl.BlockSpec((bm, K), lambda i: (i, 0)),
            pl.BlockSpec((bm, 1), lambda i: (i, 0)),
            pl.BlockSpec((1, K), lambda i: (0, 0)),
        ],
        out_specs=pl.BlockSpec((bm, K), lambda i: (i, 0)),
        out_shape=jax.ShapeDtypeStruct((M, K), jnp.float32),
    )(agg3, yt, dinv, b.reshape(1, K))


def _maxred_body(p_ref, o_ref):
    o_ref[...] = jnp.max(p_ref[...], axis=1).reshape(1, N_GRAPHS, DF)


def _tc_maxreduce(partials, dchunks):
    partials = partials.reshape(dchunks, 32, N_GRAPHS * DF)
    return pl.pallas_call(
        _maxred_body,
        grid=(dchunks,),
        in_specs=[pl.BlockSpec((1, 32, N_GRAPHS * DF), lambda f: (f, 0, 0))],
        out_specs=pl.BlockSpec((1, N_GRAPHS, DF), lambda f: (f, 0, 0)),
        out_shape=jax.ShapeDtypeStruct((dchunks, N_GRAPHS, DF), jnp.float32),
    )(partials)


def _cell_body(x_ref, w_ref, b_ref, o_ref, acc_ref, sq_ref, *, nk):
    k = pl.program_id(0)

    @pl.when(k == 0)
    def _():
        acc_ref[...] = jnp.zeros_like(acc_ref)
        sq_ref[...] = jnp.zeros_like(sq_ref)

    xb = x_ref[...]
    acc_ref[...] += jnp.dot(xb, w_ref[...], preferred_element_type=jnp.float32)
    sq_ref[...] += jnp.sum(xb * xb, axis=1, keepdims=True)

    @pl.when(k == nk - 1)
    def _():
        nrm = jnp.sqrt(sq_ref[...])
        scale = 1.0 / jnp.maximum(nrm, 1e-12)
        o_ref[...] = jnp.maximum(acc_ref[...] * scale + b_ref[...], 0.0)


def _cell_layer1(cell, w, b, bk=2048):
    M, K = cell.shape
    Kp = ((K + bk - 1) // bk) * bk
    cellp = _pad2(cell, M, Kp)
    wp = _pad2(w, Kp, w.shape[1])
    N = w.shape[1]
    nk = Kp // bk
    return pl.pallas_call(
        functools.partial(_cell_body, nk=nk),
        grid=(nk,),
        in_specs=[
            pl.BlockSpec((M, bk), lambda k: (0, k)),
            pl.BlockSpec((bk, N), lambda k: (k, 0)),
            pl.BlockSpec((1, N), lambda k: (0, 0)),
        ],
        out_specs=pl.BlockSpec((M, N), lambda k: (0, 0)),
        out_shape=jax.ShapeDtypeStruct((M, N), jnp.float32),
        scratch_shapes=[
            pltpu.VMEM((M, N), jnp.float32),
            pltpu.VMEM((M, 1), jnp.float32),
        ],
    )(cellp, wp, b.reshape(1, N))


def _head_body(g1_ref, g2_ref, c1_ref, wg1_ref, bg1_ref, wg2_ref, bg2_ref,
               wr2_ref, br2_ref, wr3_ref, br3_ref, wf1_ref, bf1_ref,
               wf2_ref, bf2_ref, wo_ref, bo_ref, o_ref, *, dchunks):
    def mm(x, w, b, relu):
        y = jnp.dot(x, w[...], preferred_element_type=jnp.float32) + b[...]
        return jnp.maximum(y, 0.0) if relu else y

    def flat(g_ref):
        return jnp.concatenate([g_ref[q] for q in range(dchunks)], axis=1)

    v1 = mm(mm(flat(g1_ref), wg1_ref, bg1_ref, True), wg2_ref, bg2_ref, False)
    v2 = mm(mm(flat(g2_ref), wg1_ref, bg1_ref, True), wg2_ref, bg2_ref, False)
    c = mm(mm(c1_ref[...], wr2_ref, br2_ref, True), wr3_ref, br3_ref, False)
    xc = jnp.concatenate([v1, v2, c], axis=1)
    h = mm(mm(xc, wf1_ref, bf1_ref, True), wf2_ref, bf2_ref, True)
    o_ref[...] = mm(h, wo_ref, bo_ref, False)


def _head(g1, g2, c1, wg1, bg1, wg2, bg2, wr2, br2, wr3, br3,
          wf1, bf1, wf2, bf2, wo, bo):
    n_out = wo.shape[1]
    dchunks = g1.shape[0]
    args = [g1, g2, c1,
            wg1, bg1.reshape(1, -1), wg2, bg2.reshape(1, -1),
            wr2, br2.reshape(1, -1), wr3, br3.reshape(1, -1),
            wf1, bf1.reshape(1, -1), wf2, bf2.reshape(1, -1),
            wo, bo.reshape(1, -1)]
    return pl.pallas_call(
        functools.partial(_head_body, dchunks=dchunks),
        out_shape=jax.ShapeDtypeStruct((N_GRAPHS, n_out), jnp.float32),
    )(*args)


def kernel(x1, edge_index1, batch1, cell, x2, edge_index2, batch2,
           W_c1, b_c1, W_c2, b_c2, W_c3, b_c3, W_g1, b_g1, W_g2, b_g2,
           W_r1, b_r1, W_r2, b_r2, W_r3, b_r3, W_f1, b_f1, W_f2, b_f2,
           W_o, b_o):
    wc1 = _pad2(W_c1, 128, 80)
    wc2 = _pad2(W_c2, 80, 160)
    wc3 = _pad2(W_c3, 160, 320)
    bc1 = _pad1(b_c1, 80)
    bc2 = _pad1(b_c2, 160)
    bc3 = _pad1(b_c3, 320)
    wg1 = _pad2(W_g1, 320, 160)
    bg1 = _pad1(b_g1, 160)
    wg2 = _pad2(W_g2, 160, 128)

    cols = jnp.concatenate([edge_index1[1], edge_index2[1]])
    degacc = _sc_deg(cols)
    dinvs = _tc_dinv(degacc)

    def branch(x, ei, batch, dinv):
        xp = _pad2(x, N_NODES, 128)
        row, col = ei[0], ei[1]
        yt1 = _mm_scale(xp, wc1, dinv)
        agg1 = _sc_agg[5](yt1.reshape(N_NODES * 5, DF), row, col)
        yt2 = _gcn_mm(agg1, yt1, dinv, bc1, wc2)
        agg2 = _sc_agg[10](yt2.reshape(N_NODES * 10, DF), row, col)
        yt3 = _gcn_mm(agg2, yt2, dinv, bc2, wc3)
        agg3 = _sc_agg[20](yt3.reshape(N_NODES * 20, DF), row, col)
        h3 = _gcn_last(agg3, yt3, dinv, bc3)
        gp = _sc_segmax[20](h3.reshape(N_NODES * 20, DF), batch)
        return _tc_maxreduce(gp, 20)

    g1 = branch(x1, edge_index1, batch1, dinvs[0])
    g2 = branch(x2, edge_index2, batch2, dinvs[1])

    c1 = _cell_layer1(cell, W_r1, b_r1)

    return _head(g1, g2, c1,
                 wg1, bg1, wg2, b_g2,
                 W_r2, b_r2, W_r3, b_r3,
                 W_f1, b_f1, W_f2, b_f2,
                 W_o, b_o)
